# baseline (speedup 1.0000x reference)
import functools

import jax
import jax.numpy as jnp
from jax import lax
from jax.experimental import pallas as pl
from jax.experimental.pallas import tpu as pltpu
from jax.experimental.pallas import tpu_sc as plsc

_NC = 2
_NS = 16
_NW = _NC * _NS
_CHUNK = 128
_LANES = 16
_PIECE = 16
_NMAX = 1000000 - 16


@functools.cache
def _build(B, E, S):
    D = E + S
    bpw = B // _NW
    nchunk = bpw // _CHUNK
    npiece = bpw // _PIECE
    groups_per_chunk = _CHUNK // _LANES

    mesh = plsc.VectorSubcoreMesh(core_axis_name="c", subcore_axis_name="s")

    @functools.partial(
        pl.kernel,
        out_type=jax.ShapeDtypeStruct((B, D), jnp.float32),
        mesh=mesh,
        scratch_types=[
            pltpu.VMEM((nchunk, _CHUNK), jnp.int32),
            pltpu.VMEM((_PIECE, 16, E), jnp.float32),
            pltpu.VMEM((bpw, D), jnp.float32),
            pltpu.VMEM((_LANES,), jnp.float32),
            pltpu.SemaphoreType.DMA,
        ],
    )
    def k(idx_hbm, table_hbm, share_hbm, out_hbm,
          idx_v, blocks_v, out_v, share_v, sem):
        cid = lax.axis_index("c")
        sid = lax.axis_index("s")
        wid = sid * _NC + cid
        base = wid * bpw
        pltpu.sync_copy(idx_hbm.at[wid], idx_v)
        pltpu.sync_copy(share_hbm, share_v)
        share_vec = share_v[...]

        def fill(t, _):
            for u in range(8):
                out_v[t * 8 + u, pl.ds(D - _LANES, _LANES)] = share_vec
            return 0

        lax.fori_loop(0, bpw // 8, fill, 0)

        def piece(p, _):
            j = p // groups_per_chunk
            r = (p % groups_per_chunk) * _LANES
            vec = idx_v[j, pl.ds(r, _LANES)]
            copies = []
            starts = []
            for u in range(_PIECE):
                start = pl.multiple_of(
                    jnp.minimum(vec[u] & ~jnp.int32(7), _NMAX), 8
                )
                copies.append(
                    pltpu.async_copy(
                        table_hbm.at[pl.ds(start, 16)], blocks_v.at[u], sem
                    )
                )
                starts.append(start)
            for cp in copies:
                cp.wait()
            for u in range(_PIECE):
                sub = vec[u] - starts[u]
                row = p * _PIECE + u
                for c in (0, 16, 32, 40):
                    out_v[row, pl.ds(c, _LANES)] = (
                        blocks_v[u, sub, pl.ds(c, _LANES)]
                    )
            return 0

        lax.fori_loop(0, npiece, piece, 0)

        pltpu.sync_copy(out_v, out_hbm.at[pl.ds(base, bpw)])

    return k


def kernel(x, embed_weight, share):
    B = x.shape[0]
    E = embed_weight.shape[1]
    S = share.shape[-1]
    idx3d = x.astype(jnp.int32).reshape(_NW, B // _NW // _CHUNK, _CHUNK)
    share16 = jnp.concatenate(
        [jnp.zeros((_LANES - S,), jnp.float32), share.reshape(S)]
    )
    out = _build(B, E, S)(idx3d, embed_weight, share16)
    return out.reshape(B, 1, E + S)


# speedup vs baseline: 1.1023x; 1.1023x over previous
import functools

import jax
import jax.numpy as jnp
from jax import lax
from jax.experimental import pallas as pl
from jax.experimental.pallas import tpu as pltpu
from jax.experimental.pallas import tpu_sc as plsc

_NC = 2
_NS = 16
_NW = _NC * _NS
_CHUNK = 128
_LANES = 16
_PIECE = 16
_GPP = _PIECE // _LANES


@functools.cache
def _build(B, E, S):
    D = E + S
    bpw = B // _NW
    nchunk = bpw // _CHUNK
    npiece = bpw // _PIECE
    groups_per_chunk = _CHUNK // _LANES

    mesh = plsc.VectorSubcoreMesh(core_axis_name="c", subcore_axis_name="s")

    @functools.partial(
        pl.kernel,
        out_type=jax.ShapeDtypeStruct((B, D), jnp.float32),
        mesh=mesh,
        scratch_types=[
            pltpu.VMEM((nchunk, _CHUNK), jnp.int32),
            pltpu.VMEM((2, _PIECE, 8, E), jnp.float32),
            pltpu.VMEM((bpw, D), jnp.float32),
            pltpu.VMEM((_LANES,), jnp.float32),
            pltpu.SemaphoreType.DMA,
            pltpu.SemaphoreType.DMA,
        ],
    )
    def k(idx_hbm, table_hbm, share_hbm, out_hbm,
          idx_v, blocks_v, out_v, share_v, sem0, sem1):
        cid = lax.axis_index("c")
        sid = lax.axis_index("s")
        wid = sid * _NC + cid
        base = wid * bpw
        pltpu.sync_copy(idx_hbm.at[wid], idx_v)
        pltpu.sync_copy(share_hbm, share_v)
        sems = (sem0, sem1)

        def vecs(p):
            flat = p * _PIECE
            j = flat // _CHUNK
            r = flat % _CHUNK
            return [
                idx_v[j, pl.ds(r + g * _LANES, _LANES)] for g in range(_GPP)
            ]

        def fire(p, buf):
            sem = sems[buf]
            for g, vec in enumerate(vecs(p)):
                for u in range(_LANES):
                    start = pl.multiple_of(vec[u] & ~jnp.int32(7), 8)
                    pltpu.async_copy(
                        table_hbm.at[pl.ds(start, 8)],
                        blocks_v.at[buf, g * _LANES + u],
                        sem,
                    )

        def drain(p, buf):
            sem = sems[buf]
            for u in range(_PIECE):
                pltpu.make_async_copy(
                    table_hbm.at[pl.ds(0, 8)], blocks_v.at[buf, u], sem
                ).wait()
            for g, vec in enumerate(vecs(p)):
                for u in range(_LANES):
                    sub = vec[u] & 7
                    row = p * _PIECE + g * _LANES + u
                    for c in (0, 16, 32, 40):
                        out_v[row, pl.ds(c, _LANES)] = (
                            blocks_v[buf, g * _LANES + u, sub, pl.ds(c, _LANES)]
                        )

        share_vec = share_v[...]

        def fill(t, _):
            for u in range(8):
                out_v[t * 8 + u, pl.ds(D - _LANES, _LANES)] = share_vec
            return 0

        fire(0, 0)
        lax.fori_loop(0, bpw // 8, fill, 0)

        def pair(t, _):
            fire(2 * t + 1, 1)
            drain(2 * t, 0)
            fire(2 * t + 2, 0)
            drain(2 * t + 1, 1)
            return 0

        lax.fori_loop(0, npiece // 2 - 1, pair, 0)
        fire(npiece - 1, 1)
        drain(npiece - 2, 0)
        drain(npiece - 1, 1)

        pltpu.sync_copy(out_v, out_hbm.at[pl.ds(base, bpw)])

    return k


def kernel(x, embed_weight, share):
    B = x.shape[0]
    E = embed_weight.shape[1]
    S = share.shape[-1]
    idx3d = x.astype(jnp.int32).reshape(_NW, B // _NW // _CHUNK, _CHUNK)
    share16 = jnp.concatenate(
        [jnp.zeros((_LANES - S,), jnp.float32), share.reshape(S)]
    )
    out = _build(B, E, S)(idx3d, embed_weight, share16)
    return out.reshape(B, 1, E + S)
